# baseline (speedup 1.0000x reference)
import jax
import jax.numpy as jnp
from jax import lax
from jax.experimental import pallas as pl
from jax.experimental.pallas import tpu as pltpu

N = 10000
DEG = 32
D_IN = 128
D_OUT = 128
BLOCK_R = 320


def _fused_body(feat_ref, emb_ref, w_ref, b_ref, out_ref):
    agg = jnp.sum(emb_ref[...], axis=1) * (1.0 / DEG)
    x = feat_ref[...] + agg
    y = lax.dot_general(
        x, w_ref[...], (((1,), (0,)), ((), ())),
        preferred_element_type=jnp.float32)
    out_ref[...] = jnp.maximum(y + b_ref[...], 0.0)


@jax.jit
def kernel(features, embedding_look_up, kernel, bias_weights):
    bias2d = bias_weights.reshape(1, D_OUT)
    return pl.pallas_call(
        _fused_body,
        grid=(-(-N // BLOCK_R),),
        in_specs=[
            pl.BlockSpec((BLOCK_R, D_IN), lambda i: (i, 0)),
            pl.BlockSpec((BLOCK_R, DEG, D_IN), lambda i: (i, 0, 0)),
            pl.BlockSpec((D_IN, D_OUT), lambda i: (0, 0)),
            pl.BlockSpec((1, D_OUT), lambda i: (0, 0)),
        ],
        out_specs=pl.BlockSpec((BLOCK_R, D_OUT), lambda i: (i, 0)),
        out_shape=jax.ShapeDtypeStruct((N, D_OUT), jnp.float32),
        compiler_params=pltpu.CompilerParams(
            dimension_semantics=("arbitrary",),
        ),
    )(features, embedding_look_up, kernel, bias2d)


# speedup vs baseline: 1.0482x; 1.0482x over previous
import jax
import jax.numpy as jnp
from jax import lax
from jax.experimental import pallas as pl
from jax.experimental.pallas import tpu as pltpu

N = 10000
DEG = 32
D_IN = 128
D_OUT = 128
BLOCK_R = 400


def _fused_body(feat_ref, emb_ref, w_ref, b_ref, out_ref):
    agg = jnp.sum(emb_ref[...], axis=1) * (1.0 / DEG)
    x = feat_ref[...] + agg
    y = lax.dot_general(
        x, w_ref[...], (((1,), (0,)), ((), ())),
        preferred_element_type=jnp.float32)
    out_ref[...] = jnp.maximum(y + b_ref[...], 0.0)


@jax.jit
def kernel(features, embedding_look_up, kernel, bias_weights):
    bias2d = bias_weights.reshape(1, D_OUT)
    return pl.pallas_call(
        _fused_body,
        grid=(-(-N // BLOCK_R),),
        in_specs=[
            pl.BlockSpec((BLOCK_R, D_IN), lambda i: (i, 0)),
            pl.BlockSpec((BLOCK_R, DEG, D_IN), lambda i: (i, 0, 0)),
            pl.BlockSpec((D_IN, D_OUT), lambda i: (0, 0)),
            pl.BlockSpec((1, D_OUT), lambda i: (0, 0)),
        ],
        out_specs=pl.BlockSpec((BLOCK_R, D_OUT), lambda i: (i, 0)),
        out_shape=jax.ShapeDtypeStruct((N, D_OUT), jnp.float32),
        compiler_params=pltpu.CompilerParams(
            dimension_semantics=("arbitrary",),
        ),
    )(features, embedding_look_up, kernel, bias2d)
